# Initial kernel scaffold; baseline (speedup 1.0000x reference)
import jax
import jax.numpy as jnp
from jax.experimental import pallas as pl


def kernel(x, edge_index, W1, b1, W2, b2, W3, b3):
    raise NotImplementedError("write your pallas kernel here")


# speedup vs baseline: 3.5581x; 3.5581x over previous
import functools

import jax
import jax.numpy as jnp
from jax import lax
from jax.experimental import pallas as pl
from jax.experimental.pallas import tpu as pltpu
from jax.experimental.pallas import tpu_sc as plsc

_N = 10000
_E = 160000
_DIN = 256
_DH = 512

_NSC = 2
_NTILE = 16
_WIN = 128
_CW = 128
_NPT = 640
_NACC = _NTILE * _NPT
_NWT = 88
_NBW = 8
_NBLK = _NWT // _NBW
_SELFW = 6
_EPT = _NWT * _WIN
_EPAD = _NTILE * _EPT
_NDW = _EPAD // (_NSC * _NTILE * _WIN)
_MT = 400

_mesh = plsc.VectorSubcoreMesh(
    core_axis_name="c", subcore_axis_name="s",
    num_cores=_NSC, num_subcores=_NTILE)


def _fill_1d(ref, n, vec16):
    def body(i, _):
        ref[pl.ds(i * 16, 16)] = vec16
        return 0
    lax.fori_loop(0, n // 16, body, 0)


def _zero_2d(ref, rows, cols):
    z16 = jnp.zeros((16,), jnp.float32)
    def body(i, _):
        for k in range(cols // 16):
            ref[i, pl.ds(k * 16, 16)] = z16
        return 0
    lax.fori_loop(0, rows, body, 0)


def _deg_body(dst_hbm, out_hbm, dst_v, buf_v, ones_v, deg_sh):
    c = lax.axis_index("c")
    s = lax.axis_index("s")
    wid = c * _NTILE + s
    pltpu.sync_copy(dst_hbm.at[wid], dst_v)
    _fill_1d(ones_v, _WIN, jnp.ones((16,), jnp.float32))
    _fill_1d(buf_v, _NPT, jnp.zeros((16,), jnp.float32))
    pltpu.sync_copy(buf_v, deg_sh.at[pl.ds(s * _NPT, _NPT)])
    plsc.subcore_barrier()

    def scat(j, _):
        pltpu.sync_copy(ones_v, deg_sh.at[dst_v.at[j]], add=True)
        return 0
    lax.fori_loop(0, _NDW, scat, 0)
    plsc.subcore_barrier()
    pltpu.sync_copy(deg_sh.at[pl.ds(s * _NPT, _NPT)], buf_v)
    pltpu.sync_copy(buf_v, out_hbm.at[c, pl.ds(s * _NPT, _NPT)])


def _sc_deg(dst2d):
    kfn = pl.kernel(
        _deg_body,
        out_type=jax.ShapeDtypeStruct((_NSC, _NACC), jnp.float32),
        mesh=_mesh,
        scratch_types=[
            pltpu.VMEM((_NDW, _WIN), jnp.int32),

            pltpu.VMEM((_NPT,), jnp.float32),
            pltpu.VMEM((_WIN,), jnp.float32),
            pltpu.VMEM_SHARED((_NACC,), jnp.float32),
        ])
    return kfn(dst2d)


def _prop_body(C, src_hbm, dst_hbm, hs_hbm, g_hbm,
               sblk, dblk, r0, r1, acc_sh, gs0, gs1, ss0, ss1):
    c = lax.axis_index("c")
    s = lax.axis_index("s")

    def task(t, _):
        chunk = c + _NSC * t
        base = chunk * _N

        def load_blk(b):
            pltpu.sync_copy(src_hbm.at[s, pl.ds(b * _NBW, _NBW)], sblk)
            pltpu.sync_copy(dst_hbm.at[s, pl.ds(b * _NBW, _NBW)], dblk)
            def ob(i, _):
                for k in range(_WIN // 16):
                    sl = pl.ds(k * 16, 16)
                    sblk[i, sl] = sblk[i, sl] + base
                return 0
            lax.fori_loop(0, _NBW, ob, 0)

        load_blk(0)
        for j in range(_SELFW):
            pltpu.sync_copy(hs_hbm.at[sblk.at[j]], r0)
            pltpu.sync_copy(r0, acc_sh.at[dblk.at[j]])
        plsc.subcore_barrier()
        for j in range(_SELFW, _NBW):
            pltpu.sync_copy(hs_hbm.at[sblk.at[j]], r0)
            pltpu.sync_copy(r0, acc_sh.at[dblk.at[j]], add=True)

        def blk(b, _):
            load_blk(b)
            pltpu.async_copy(hs_hbm.at[sblk.at[0]], r0, gs0)

            def pair(i, _):
                j0 = 2 * i
                j1 = j0 + 1
                pltpu.make_async_copy(hs_hbm.at[sblk.at[j0]], r0, gs0).wait()
                pltpu.async_copy(r0, acc_sh.at[dblk.at[j0]], ss0, add=True)

                @pl.when(i > 0)
                def _w1():
                    pltpu.make_async_copy(
                        r1, acc_sh.at[dblk.at[j1]], ss1).wait()

                pltpu.async_copy(hs_hbm.at[sblk.at[j1]], r1, gs1)
                pltpu.make_async_copy(hs_hbm.at[sblk.at[j1]], r1, gs1).wait()
                pltpu.async_copy(r1, acc_sh.at[dblk.at[j1]], ss1, add=True)
                pltpu.make_async_copy(r0, acc_sh.at[dblk.at[j0]], ss0).wait()

                @pl.when(i < _NBW // 2 - 1)
                def _g0():
                    pltpu.async_copy(hs_hbm.at[sblk.at[j0 + 2]], r0, gs0)
                return 0
            lax.fori_loop(0, _NBW // 2, pair, 0)
            pltpu.make_async_copy(r1, acc_sh.at[dblk.at[0]], ss1).wait()
            return 0
        lax.fori_loop(1, _NBLK, blk, 0)
        plsc.subcore_barrier()

        for k in range(_NPT // _WIN):
            row = s * _NPT + k * _WIN
            pltpu.sync_copy(acc_sh.at[pl.ds(row, _WIN)], r0)
            pltpu.sync_copy(r0, g_hbm.at[pl.ds(chunk * _NACC + row, _WIN)])
        plsc.subcore_barrier()
        return 0

    lax.fori_loop(0, C // _NSC, task, 0)


def _sc_prop(C, src3d, dst3d, hs_all):
    kfn = pl.kernel(
        functools.partial(_prop_body, C),
        out_type=jax.ShapeDtypeStruct((C * _NACC, _CW), jnp.float32),
        mesh=_mesh,
        scratch_types=[
            pltpu.VMEM((_NBW, _WIN), jnp.int32),
            pltpu.VMEM((_NBW, _WIN), jnp.int32),
            pltpu.VMEM((_WIN, _CW), jnp.float32),
            pltpu.VMEM((_WIN, _CW), jnp.float32),
            pltpu.VMEM_SHARED((_NACC, _CW), jnp.float32),
            pltpu.SemaphoreType.DMA,
            pltpu.SemaphoreType.DMA,
            pltpu.SemaphoreType.DMA,
            pltpu.SemaphoreType.DMA,
        ])
    return kfn(src3d, dst3d, hs_all)


def _prep_body(deg_ref, x_ref, dinv_ref, s0_ref):
    d = deg_ref[:, 0] + deg_ref[:, 1]
    dinv = lax.rsqrt(d)
    dinv_ref[...] = dinv[:, None]
    sx = x_ref[...] * dinv[:, None]
    for ci in range(_DIN // _CW):
        s0_ref[ci, :, :] = sx[:, ci * _CW:(ci + 1) * _CW]


def _tc_prep(deg2, x):
    grid = (_N // _MT,)
    return pl.pallas_call(
        _prep_body,
        grid=grid,
        in_specs=[
            pl.BlockSpec((_MT, _NSC), lambda m: (m, 0)),
            pl.BlockSpec((_MT, _DIN), lambda m: (m, 0)),
        ],
        out_specs=[
            pl.BlockSpec((_MT, 1), lambda m: (m, 0)),
            pl.BlockSpec((_DIN // _CW, _MT, _CW), lambda m: (0, m, 0)),
        ],
        out_shape=[
            jax.ShapeDtypeStruct((_N, 1), jnp.float32),
            jax.ShapeDtypeStruct((_DIN // _CW, _N, _CW), jnp.float32),
        ],
    )(deg2.T, x)


def _mm_body(C_in, relu, chunk_out, g_ref, dinv_ref, w_ref, b_ref, o_ref):
    dinv = dinv_ref[...]
    acc = jnp.zeros((_MT, _DH), jnp.float32)
    for ci in range(C_in):
        a = g_ref[ci] * dinv
        acc = acc + jnp.dot(a, w_ref[pl.ds(ci * _CW, _CW), :],
                            preferred_element_type=jnp.float32)
    z = acc + b_ref[...]
    if relu:
        z = jnp.maximum(z, 0.0)
    if chunk_out:
        zs = z * dinv
        for co in range(_DH // _CW):
            o_ref[co, :, :] = zs[:, co * _CW:(co + 1) * _CW]
    else:
        o_ref[...] = z


def _tc_mm(g_all, dinv, w, b, C_in, relu, chunk_out):
    grid = (_N // _MT,)
    g3 = g_all.reshape(C_in, _NACC, _CW)
    if chunk_out:
        out_shape = jax.ShapeDtypeStruct((_DH // _CW, _N, _CW), jnp.float32)
        out_spec = pl.BlockSpec((_DH // _CW, _MT, _CW), lambda m: (0, m, 0))
    else:
        out_shape = jax.ShapeDtypeStruct((_N, _DH), jnp.float32)
        out_spec = pl.BlockSpec((_MT, _DH), lambda m: (m, 0))
    return pl.pallas_call(
        functools.partial(_mm_body, C_in, relu, chunk_out),
        grid=grid,
        in_specs=[
            pl.BlockSpec((C_in, _MT, _CW), lambda m: (0, m, 0)),
            pl.BlockSpec((_MT, 1), lambda m: (m, 0)),
            pl.BlockSpec((C_in * _CW, _DH), lambda m: (0, 0)),
            pl.BlockSpec((1, _DH), lambda m: (0, 0)),
        ],
        out_specs=out_spec,
        out_shape=out_shape,
    )(g3, dinv, w, b.reshape(1, _DH))


def kernel(x, edge_index, W1, b1, W2, b2, W3, b3):
    njunk = _NACC - _N
    ar = jnp.arange(_NACC, dtype=jnp.int32)
    self_src = jnp.where(ar < _N, ar, 0).reshape(_NTILE, _NPT)
    self_dst = ar.reshape(_NTILE, _NPT)
    njo = _NTILE * (_SELFW * _WIN - _NPT)
    jo = jnp.arange(njo, dtype=jnp.int32)
    jo_src = jnp.zeros((njo,), jnp.int32).reshape(_NTILE, -1)
    jo_dst = (_N + jo % njunk).reshape(_NTILE, -1)
    nadd = _NTILE * (_EPT - _SELFW * _WIN)
    pad = nadd - _E
    padi = jnp.arange(pad, dtype=jnp.int32)
    add_src = jnp.concatenate(
        [edge_index[0], jnp.zeros((pad,), jnp.int32)]).reshape(_NTILE, -1)
    add_dst = jnp.concatenate(
        [edge_index[1], _N + padi % njunk]).reshape(_NTILE, -1)
    src16 = jnp.concatenate([self_src, jo_src, add_src], axis=1).reshape(
        _NTILE, _NWT, _WIN)
    dst16 = jnp.concatenate([self_dst, jo_dst, add_dst], axis=1).reshape(
        _NTILE, _NWT, _WIN)
    dst32 = dst16.reshape(_NSC * _NTILE, _NDW, _WIN)

    deg2 = _sc_deg(dst32)
    dinv, s0 = _tc_prep(deg2, x)

    c1 = _DIN // _CW
    c2 = _DH // _CW
    g1 = _sc_prop(c1, src16, dst16, s0.reshape(c1 * _N, _CW))
    s1 = _tc_mm(g1, dinv, W1, b1, C_in=c1, relu=True, chunk_out=True)
    g2 = _sc_prop(c2, src16, dst16, s1.reshape(c2 * _N, _CW))
    s2 = _tc_mm(g2, dinv, W2, b2, C_in=c2, relu=True, chunk_out=True)
    g3 = _sc_prop(c2, src16, dst16, s2.reshape(c2 * _N, _CW))
    out = _tc_mm(g3, dinv, W3, b3, C_in=c2, relu=False, chunk_out=False)
    return out
